# Initial kernel scaffold; baseline (speedup 1.0000x reference)
import functools

import jax
import jax.numpy as jnp
from jax import lax
from jax.experimental import pallas as pl
from jax.experimental.pallas import tpu as pltpu
from jax.experimental.pallas import tpu_sc as plsc

N = 10000
D = 128
NC = 2
NS = 16
NW = NC * NS
NPAD = 10240
ROWS_PER_TILE = NPAD // NS
CHUNK = 128
BLK = 2048
GRID = 5


def _mesh():
  return plsc.VectorSubcoreMesh(
      core_axis_name="c", subcore_axis_name="s", num_cores=NC, num_subcores=NS
  )


def _make_deg_kernel(e_per_tile):
  n_vec = e_per_tile // 16
  out_slice = NPAD // NS

  @functools.partial(
      pl.kernel,
      out_type=jax.ShapeDtypeStruct((NC, NPAD), jnp.float32),
      mesh=_mesh(),
      scratch_types=[
          pltpu.VMEM((e_per_tile,), jnp.int32),
          pltpu.VMEM((NPAD,), jnp.float32),
          pltpu.VMEM((out_slice,), jnp.float32),
          pltpu.VMEM((out_slice,), jnp.float32),
          pltpu.VMEM_SHARED((NS, NPAD), jnp.float32),
      ],
  )
  def deg_kernel(dst_hbm, deg_out, dst_v, hist_v, acc_v, tmp_v, shared):
    c = lax.axis_index("c")
    s = lax.axis_index("s")
    wid = c * NS + s
    pltpu.sync_copy(dst_hbm.at[pl.ds(wid * e_per_tile, e_per_tile)], dst_v)

    zeros16 = jnp.zeros((16,), jnp.float32)
    ones16 = jnp.ones((16,), jnp.float32)

    def zero_body(i, carry):
      hist_v[pl.ds(i * 16, 16)] = zeros16
      return carry

    lax.fori_loop(0, NPAD // 16, zero_body, 0)

    def count_body(i, carry):
      idx = dst_v[pl.ds(i * 16, 16)]
      plsc.addupdate_scatter(hist_v, [idx], ones16)
      return carry

    lax.fori_loop(0, n_vec, count_body, 0)

    pltpu.sync_copy(hist_v, shared.at[s])
    plsc.subcore_barrier()

    base = s * out_slice

    def zero_acc(i, carry):
      acc_v[pl.ds(i * 16, 16)] = zeros16
      return carry

    lax.fori_loop(0, out_slice // 16, zero_acc, 0)

    def peer_body(j, carry):
      pltpu.sync_copy(shared.at[j, pl.ds(base, out_slice)], tmp_v)

      def add_body(k, c2):
        sl = pl.ds(k * 16, 16)
        acc_v[sl] = acc_v[sl] + tmp_v[sl]
        return c2

      lax.fori_loop(0, out_slice // 16, add_body, 0)
      return carry

    lax.fori_loop(0, NS, peer_body, 0)
    pltpu.sync_copy(acc_v, deg_out.at[c, pl.ds(base, out_slice)])

  return deg_kernel


def _matmul_body(x_ref, w_ref, deg_ref, g_ref):
  dsum = deg_ref[0, 0, :] + deg_ref[1, 0, :] + 1.0
  dinv = lax.rsqrt(dsum)
  h = jnp.dot(x_ref[...], w_ref[...], preferred_element_type=jnp.float32)
  g_ref[...] = h * dinv[:, None]


def _matmul_scaled(x, w, deg):
  deg3 = deg.reshape(NC, GRID, BLK)
  return pl.pallas_call(
      _matmul_body,
      grid=(GRID,),
      in_specs=[
          pl.BlockSpec((BLK, D), lambda i: (i, 0)),
          pl.BlockSpec((D, D), lambda i: (0, 0)),
          pl.BlockSpec((NC, 1, BLK), lambda i: (0, i, 0)),
      ],
      out_specs=pl.BlockSpec((BLK, D), lambda i: (i, 0)),
      out_shape=jax.ShapeDtypeStruct((N, D), jnp.float32),
  )(x, w, deg3)


def _make_agg_kernel(chunks_per_tile):
  @functools.partial(
      pl.kernel,
      out_type=jax.ShapeDtypeStruct((NC, NPAD, D), jnp.float32),
      mesh=_mesh(),
      scratch_types=[
          pltpu.VMEM((chunks_per_tile, CHUNK), jnp.int32),
          pltpu.VMEM((chunks_per_tile, CHUNK), jnp.int32),
          pltpu.VMEM((CHUNK, D), jnp.float32),
          pltpu.VMEM((CHUNK, D), jnp.float32),
          pltpu.VMEM_SHARED((NPAD, D), jnp.float32),
          pltpu.SemaphoreType.DMA,
          pltpu.SemaphoreType.DMA,
      ],
  )
  def agg_kernel(g_hbm, src_hbm, dst_hbm, zero_hbm, acc_out,
                 src_v, dst_v, rows_a, rows_b, acc_sh, sem_a, sem_b):
    c = lax.axis_index("c")
    s = lax.axis_index("s")
    wid = c * NS + s

    pltpu.sync_copy(zero_hbm,
                    acc_sh.at[pl.ds(s * ROWS_PER_TILE, ROWS_PER_TILE)])

    pltpu.sync_copy(src_hbm.at[wid], src_v)
    pltpu.sync_copy(dst_hbm.at[wid], dst_v)
    plsc.subcore_barrier()

    pltpu.async_copy(g_hbm.at[src_v.at[0]], rows_a, sem_a)

    def chunk_body(j, carry):
      use_a = lax.rem(j, 2) == 0
      nxt = j + 1

      @pl.when(nxt < chunks_per_tile)
      def _start_next():
        @pl.when(use_a)
        def _():
          pltpu.async_copy(g_hbm.at[src_v.at[nxt]], rows_b, sem_b)

        @pl.when(jnp.logical_not(use_a))
        def _():
          pltpu.async_copy(g_hbm.at[src_v.at[nxt]], rows_a, sem_a)

      @pl.when(use_a)
      def _drain_a():
        pltpu.make_async_copy(g_hbm.at[src_v.at[0]], rows_a, sem_a).wait()
        pltpu.sync_copy(rows_a, acc_sh.at[dst_v.at[j]], add=True)

      @pl.when(jnp.logical_not(use_a))
      def _drain_b():
        pltpu.make_async_copy(g_hbm.at[src_v.at[0]], rows_b, sem_b).wait()
        pltpu.sync_copy(rows_b, acc_sh.at[dst_v.at[j]], add=True)

      return carry

    lax.fori_loop(0, chunks_per_tile, chunk_body, 0)

    plsc.subcore_barrier()
    sl = pl.ds(s * ROWS_PER_TILE, ROWS_PER_TILE)
    pltpu.sync_copy(acc_sh.at[sl], acc_out.at[c, sl])

  return agg_kernel


def _final_body(acc_ref, g_ref, deg_ref, b_ref, out_ref):
  dsum = deg_ref[0, 0, :] + deg_ref[1, 0, :] + 1.0
  dinv = lax.rsqrt(dsum)
  tot = acc_ref[0] + acc_ref[1] + g_ref[...]
  out_ref[...] = jnp.maximum(tot * dinv[:, None] + b_ref[...][None, :], 0.0)


def _finalize(acc, g, deg, b):
  deg3 = deg.reshape(NC, GRID, BLK)
  return pl.pallas_call(
      _final_body,
      grid=(GRID,),
      in_specs=[
          pl.BlockSpec((NC, BLK, D), lambda i: (0, i, 0)),
          pl.BlockSpec((BLK, D), lambda i: (i, 0)),
          pl.BlockSpec((NC, 1, BLK), lambda i: (0, i, 0)),
          pl.BlockSpec((D,), lambda i: (0,)),
      ],
      out_specs=pl.BlockSpec((BLK, D), lambda i: (i, 0)),
      out_shape=jax.ShapeDtypeStruct((N, D), jnp.float32),
  )(acc, g, deg3, b)


def kernel(x, edge_index, W, b):
  E = edge_index.shape[1]
  chunks_per_tile = -(-E // (NW * CHUNK))
  e_pad = NW * chunks_per_tile * CHUNK

  ei = edge_index.astype(jnp.int32)
  src = jnp.concatenate([ei[0], jnp.zeros((e_pad - E,), jnp.int32)])
  dst = jnp.concatenate([ei[1], jnp.full((e_pad - E,), NPAD - 1, jnp.int32)])
  src3 = src.reshape(NW, chunks_per_tile, CHUNK)
  dst3 = dst.reshape(NW, chunks_per_tile, CHUNK)

  deg = _make_deg_kernel(e_pad // NW)(dst)
  g = _matmul_scaled(x, W, deg)
  zeros = jnp.zeros((ROWS_PER_TILE, D), jnp.float32)
  acc = _make_agg_kernel(chunks_per_tile)(g, src3, dst3, zeros)
  return _finalize(acc, g, deg, b)


# speedup vs baseline: 34.5504x; 34.5504x over previous
import functools

import jax
import jax.numpy as jnp
from jax import lax
from jax.experimental import pallas as pl
from jax.experimental.pallas import tpu as pltpu
from jax.experimental.pallas import tpu_sc as plsc

N = 10000
D = 128
NC = 2
NS = 16
NW = NC * NS
NPAD = 10240
ROWS_PER_TILE = NPAD // NS
CHUNK = 128
BLK = 2048
GRID = 5


def _mesh():
  return plsc.VectorSubcoreMesh(
      core_axis_name="c", subcore_axis_name="s", num_cores=NC, num_subcores=NS
  )


def _make_deg_kernel(e_per_tile):
  n_vec = e_per_tile // 16
  out_slice = NPAD // NS

  @functools.partial(
      pl.kernel,
      out_type=jax.ShapeDtypeStruct((NC, NPAD), jnp.float32),
      mesh=_mesh(),
      scratch_types=[
          pltpu.VMEM((e_per_tile,), jnp.int32),
          pltpu.VMEM((NPAD,), jnp.float32),
          pltpu.VMEM((out_slice,), jnp.float32),
          pltpu.VMEM((out_slice,), jnp.float32),
          pltpu.VMEM_SHARED((NS, NPAD), jnp.float32),
      ],
      compiler_params=pltpu.CompilerParams(needs_layout_passes=False),
  )
  def deg_kernel(dst_hbm, deg_out, dst_v, hist_v, acc_v, tmp_v, shared):
    c = lax.axis_index("c")
    s = lax.axis_index("s")
    wid = c * NS + s
    pltpu.sync_copy(dst_hbm.at[pl.ds(wid * e_per_tile, e_per_tile)], dst_v)

    zeros16 = jnp.zeros((16,), jnp.float32)
    ones16 = jnp.ones((16,), jnp.float32)

    def zero_body(i, carry):
      hist_v[pl.ds(i * 16, 16)] = zeros16
      return carry

    lax.fori_loop(0, NPAD // 16, zero_body, 0)

    def count_body(i, carry):
      idx = dst_v[pl.ds(i * 16, 16)]
      plsc.addupdate_scatter(hist_v, [idx], ones16)
      return carry

    lax.fori_loop(0, n_vec, count_body, 0)

    pltpu.sync_copy(hist_v, shared.at[s])
    plsc.subcore_barrier()

    base = s * out_slice

    def zero_acc(i, carry):
      acc_v[pl.ds(i * 16, 16)] = zeros16
      return carry

    lax.fori_loop(0, out_slice // 16, zero_acc, 0)

    def peer_body(j, carry):
      pltpu.sync_copy(shared.at[j, pl.ds(base, out_slice)], tmp_v)

      def add_body(k, c2):
        sl = pl.ds(k * 16, 16)
        acc_v[sl] = acc_v[sl] + tmp_v[sl]
        return c2

      lax.fori_loop(0, out_slice // 16, add_body, 0)
      return carry

    lax.fori_loop(0, NS, peer_body, 0)
    pltpu.sync_copy(acc_v, deg_out.at[c, pl.ds(base, out_slice)])

  return deg_kernel


def _matmul_body(x_ref, w_ref, deg_ref, g_ref):
  dsum = deg_ref[0, 0, :] + deg_ref[0, 1, :] + 1.0
  dinv = lax.rsqrt(dsum)
  h = jnp.dot(x_ref[...], w_ref[...], preferred_element_type=jnp.float32)
  g_ref[...] = h * dinv[:, None]


def _matmul_scaled(x, w, deg):
  deg3 = deg.reshape(NC, GRID, BLK).transpose(1, 0, 2)
  return pl.pallas_call(
      _matmul_body,
      grid=(GRID,),
      in_specs=[
          pl.BlockSpec((BLK, D), lambda i: (i, 0)),
          pl.BlockSpec((D, D), lambda i: (0, 0)),
          pl.BlockSpec((1, NC, BLK), lambda i: (i, 0, 0)),
      ],
      out_specs=pl.BlockSpec((BLK, D), lambda i: (i, 0)),
      out_shape=jax.ShapeDtypeStruct((N, D), jnp.float32),
  )(x, w, deg3)


HALF = NPAD // NC
OUT_ROWS = HALF // NS


def _make_agg_kernel(chunks_per_tile):
  @functools.partial(
      pl.kernel,
      out_type=jax.ShapeDtypeStruct((NPAD, D), jnp.float32),
      mesh=_mesh(),
      scratch_types=[
          pltpu.VMEM((chunks_per_tile, CHUNK), jnp.int32),
          pltpu.VMEM((chunks_per_tile, CHUNK), jnp.int32),
          pltpu.VMEM((CHUNK, D), jnp.float32),
          pltpu.VMEM((CHUNK, D), jnp.float32),
          pltpu.VMEM_SHARED((HALF, D), jnp.float32),
          pltpu.SemaphoreType.DMA,
          pltpu.SemaphoreType.DMA,
      ],
      compiler_params=pltpu.CompilerParams(needs_layout_passes=False),
  )
  def agg_kernel(g_hbm, src_hbm, dst_hbm, zero_hbm, acc_out,
                 src_v, dst_v, rows_a, rows_b, acc_sh, sem_a, sem_b):
    c = lax.axis_index("c")
    s = lax.axis_index("s")

    pltpu.sync_copy(zero_hbm, acc_sh.at[pl.ds(s * OUT_ROWS, OUT_ROWS)])

    pltpu.sync_copy(src_hbm.at[c, s], src_v)
    pltpu.sync_copy(dst_hbm.at[c, s], dst_v)
    plsc.subcore_barrier()

    def gather(j, rows, sem):
      idx = plsc.Indices(src_v.at[j], ignored_value=-1)
      return pltpu.async_copy(g_hbm.at[idx], rows, sem)

    def scatter_add(j, rows):
      idx = plsc.Indices(dst_v.at[j], ignored_value=-1)
      pltpu.sync_copy(rows, acc_sh.at[idx], add=True)

    gather(0, rows_a, sem_a)

    def chunk_body(j, carry):
      use_a = lax.rem(j, 2) == 0
      nxt = j + 1

      @pl.when(nxt < chunks_per_tile)
      def _start_next():
        @pl.when(use_a)
        def _():
          gather(nxt, rows_b, sem_b)

        @pl.when(jnp.logical_not(use_a))
        def _():
          gather(nxt, rows_a, sem_a)

      @pl.when(use_a)
      def _drain_a():
        pltpu.make_async_copy(g_hbm.at[src_v.at[0]], rows_a, sem_a).wait()
        scatter_add(j, rows_a)

      @pl.when(jnp.logical_not(use_a))
      def _drain_b():
        pltpu.make_async_copy(g_hbm.at[src_v.at[0]], rows_b, sem_b).wait()
        scatter_add(j, rows_b)

      return carry

    lax.fori_loop(0, chunks_per_tile, chunk_body, 0)

    plsc.subcore_barrier()
    sl = pl.ds(s * OUT_ROWS, OUT_ROWS)
    out_sl = pl.ds(c * HALF + s * OUT_ROWS, OUT_ROWS)
    pltpu.sync_copy(acc_sh.at[sl], acc_out.at[out_sl])

  return agg_kernel


def _final_body(acc_ref, g_ref, deg_ref, b_ref, out_ref):
  dsum = deg_ref[0, 0, :] + deg_ref[0, 1, :] + 1.0
  dinv = lax.rsqrt(dsum)
  tot = acc_ref[...] + g_ref[...]
  out_ref[...] = jnp.maximum(tot * dinv[:, None] + b_ref[...][None, :], 0.0)


def _finalize(acc, g, deg, b):
  deg3 = deg.reshape(NC, GRID, BLK).transpose(1, 0, 2)
  return pl.pallas_call(
      _final_body,
      grid=(GRID,),
      in_specs=[
          pl.BlockSpec((BLK, D), lambda i: (i, 0)),
          pl.BlockSpec((BLK, D), lambda i: (i, 0)),
          pl.BlockSpec((1, NC, BLK), lambda i: (i, 0, 0)),
          pl.BlockSpec((D,), lambda i: (0,)),
      ],
      out_specs=pl.BlockSpec((BLK, D), lambda i: (i, 0)),
      out_shape=jax.ShapeDtypeStruct((N, D), jnp.float32),
  )(acc, g, deg3, b)


def kernel(x, edge_index, W, b):
  E = edge_index.shape[1]
  ei = edge_index.astype(jnp.int32)

  e_deg = -(-E // (NW * 16)) * (NW * 16)
  dst_deg = jnp.concatenate(
      [ei[1], jnp.full((e_deg - E,), NPAD - 1, jnp.int32)])
  deg = _make_deg_kernel(e_deg // NW)(dst_deg)

  g = _matmul_scaled(x, W, deg)

  chunks_per_tile = -(-E // (NS * CHUNK))
  e_pad = NS * chunks_per_tile * CHUNK
  pad = jnp.full((e_pad - E,), -1, jnp.int32)
  src = jnp.concatenate([ei[0], pad])
  dst = jnp.concatenate([ei[1], pad])
  half_id = dst // HALF
  core = jnp.arange(NC, dtype=jnp.int32)[:, None]
  own = (half_id[None, :] == core) & (dst[None, :] >= 0)
  src_m = jnp.where(own, src[None, :], -1)
  dst_m = jnp.where(own, dst[None, :] - core * HALF, -1)
  src4 = src_m.reshape(NC, NS, chunks_per_tile, CHUNK)
  dst4 = dst_m.reshape(NC, NS, chunks_per_tile, CHUNK)

  zeros = jnp.zeros((OUT_ROWS, D), jnp.float32)
  acc = _make_agg_kernel(chunks_per_tile)(g, src4, dst4, zeros)
  return _finalize(acc, g, deg, b)


# speedup vs baseline: 37.2719x; 1.0788x over previous
import functools

import jax
import jax.numpy as jnp
from jax import lax
from jax.experimental import pallas as pl
from jax.experimental.pallas import tpu as pltpu
from jax.experimental.pallas import tpu_sc as plsc

N = 10000
D = 128
NC = 2
NS = 16
NW = NC * NS
NPAD = 10240
ROWS_PER_TILE = NPAD // NS
CHUNK = 128
BLK = 2048
GRID = 5


def _mesh():
  return plsc.VectorSubcoreMesh(
      core_axis_name="c", subcore_axis_name="s", num_cores=NC, num_subcores=NS
  )


def _make_deg_kernel(e_per_tile):
  n_vec = e_per_tile // 16
  out_slice = NPAD // NS

  @functools.partial(
      pl.kernel,
      out_type=jax.ShapeDtypeStruct((NC, NPAD), jnp.float32),
      mesh=_mesh(),
      scratch_types=[
          pltpu.VMEM((e_per_tile,), jnp.int32),
          pltpu.VMEM((NPAD,), jnp.float32),
          pltpu.VMEM((out_slice,), jnp.float32),
          pltpu.VMEM((out_slice,), jnp.float32),
          pltpu.VMEM_SHARED((NS, NPAD), jnp.float32),
      ],
      compiler_params=pltpu.CompilerParams(needs_layout_passes=False),
  )
  def deg_kernel(dst_hbm, deg_out, dst_v, hist_v, acc_v, tmp_v, shared):
    c = lax.axis_index("c")
    s = lax.axis_index("s")
    wid = c * NS + s
    pltpu.sync_copy(dst_hbm.at[pl.ds(wid * e_per_tile, e_per_tile)], dst_v)

    zeros16 = jnp.zeros((16,), jnp.float32)
    ones16 = jnp.ones((16,), jnp.float32)

    def zero_body(i, carry):
      hist_v[pl.ds(i * 16, 16)] = zeros16
      return carry

    lax.fori_loop(0, NPAD // 16, zero_body, 0)

    def count_body(i, carry):
      idx = dst_v[pl.ds(i * 16, 16)]
      plsc.addupdate_scatter(hist_v, [idx], ones16)
      return carry

    lax.fori_loop(0, n_vec, count_body, 0)

    pltpu.sync_copy(hist_v, shared.at[s])
    plsc.subcore_barrier()

    base = s * out_slice

    def zero_acc(i, carry):
      acc_v[pl.ds(i * 16, 16)] = zeros16
      return carry

    lax.fori_loop(0, out_slice // 16, zero_acc, 0)

    def peer_body(j, carry):
      pltpu.sync_copy(shared.at[j, pl.ds(base, out_slice)], tmp_v)

      def add_body(k, c2):
        sl = pl.ds(k * 16, 16)
        acc_v[sl] = acc_v[sl] + tmp_v[sl]
        return c2

      lax.fori_loop(0, out_slice // 16, add_body, 0)
      return carry

    lax.fori_loop(0, NS, peer_body, 0)
    pltpu.sync_copy(acc_v, deg_out.at[c, pl.ds(base, out_slice)])

  return deg_kernel


def _matmul_body(x_ref, w_ref, deg_ref, g_ref):
  dsum = deg_ref[0, 0, :] + deg_ref[0, 1, :] + 1.0
  dinv = lax.rsqrt(dsum)
  h = jnp.dot(x_ref[...], w_ref[...], preferred_element_type=jnp.float32)
  g_ref[...] = h * dinv[:, None]


def _matmul_scaled(x, w, deg):
  deg3 = deg.reshape(NC, GRID, BLK).transpose(1, 0, 2)
  return pl.pallas_call(
      _matmul_body,
      grid=(GRID,),
      in_specs=[
          pl.BlockSpec((BLK, D), lambda i: (i, 0)),
          pl.BlockSpec((D, D), lambda i: (0, 0)),
          pl.BlockSpec((1, NC, BLK), lambda i: (i, 0, 0)),
      ],
      out_specs=pl.BlockSpec((BLK, D), lambda i: (i, 0)),
      out_shape=jax.ShapeDtypeStruct((N, D), jnp.float32),
  )(x, w, deg3)


HALF = NPAD // NC
OUT_ROWS = HALF // NS


NBUF = 3


def _make_agg_kernel(chunks_per_tile):
  assert chunks_per_tile % NBUF == 0
  ngroups = chunks_per_tile // NBUF

  @functools.partial(
      pl.kernel,
      out_type=jax.ShapeDtypeStruct((NPAD, D), jnp.float32),
      mesh=_mesh(),
      scratch_types=[
          pltpu.VMEM((chunks_per_tile, CHUNK), jnp.int32),
          pltpu.VMEM((chunks_per_tile, CHUNK), jnp.int32),
          pltpu.VMEM((NBUF, CHUNK, D), jnp.float32),
          pltpu.SemaphoreType.DMA((NBUF,)),
          pltpu.SemaphoreType.DMA((NBUF,)),
          pltpu.VMEM_SHARED((HALF, D), jnp.float32),
      ],
      compiler_params=pltpu.CompilerParams(needs_layout_passes=False),
  )
  def agg_kernel(g_hbm, src_hbm, dst_hbm, zero_hbm, acc_out,
                 src_v, dst_v, rows, gsem, ssem, acc_sh):
    c = lax.axis_index("c")
    s = lax.axis_index("s")

    pltpu.sync_copy(zero_hbm, acc_sh.at[pl.ds(s * OUT_ROWS, OUT_ROWS)])

    pltpu.sync_copy(src_hbm.at[c, s], src_v)
    pltpu.sync_copy(dst_hbm.at[c, s], dst_v)
    plsc.subcore_barrier()

    def gather(j, b):
      idx = plsc.Indices(src_v.at[j], ignored_value=-1)
      pltpu.async_copy(g_hbm.at[idx], rows.at[b], gsem.at[b])

    def wait_gather(j, b):
      idx = plsc.Indices(src_v.at[j], ignored_value=-1)
      pltpu.make_async_copy(g_hbm.at[idx], rows.at[b], gsem.at[b]).wait()

    def scatter_add(j, b):
      idx = plsc.Indices(dst_v.at[j], ignored_value=-1)
      pltpu.async_copy(rows.at[b], acc_sh.at[idx], ssem.at[b], add=True)

    def wait_scatter(j, b):
      idx = plsc.Indices(dst_v.at[j], ignored_value=-1)
      pltpu.make_async_copy(rows.at[b], acc_sh.at[idx], ssem.at[b]).wait()

    def prime_body(b, carry):
      gather(b, b)
      return carry

    lax.fori_loop(0, NBUF, prime_body, 0)

    def group_body(grp, carry):
      base = grp * NBUF

      def fire_body(b, carry2):
        wait_gather(base + b, b)
        scatter_add(base + b, b)
        return carry2

      lax.fori_loop(0, NBUF, fire_body, 0)

      def refill_body(b, carry2):
        j = base + NBUF + b

        @pl.when(j < chunks_per_tile)
        def _refill():
          wait_scatter(base + b, b)
          gather(j, b)

        return carry2

      lax.fori_loop(0, NBUF, refill_body, 0)
      return carry

    lax.fori_loop(0, ngroups, group_body, 0)

    def drain_body(b, carry):
      wait_scatter((ngroups - 1) * NBUF + b, b)
      return carry

    lax.fori_loop(0, NBUF, drain_body, 0)

    plsc.subcore_barrier()
    sl = pl.ds(s * OUT_ROWS, OUT_ROWS)
    out_sl = pl.ds(c * HALF + s * OUT_ROWS, OUT_ROWS)
    pltpu.sync_copy(acc_sh.at[sl], acc_out.at[out_sl])

  return agg_kernel


def _final_body(acc_ref, g_ref, deg_ref, b_ref, out_ref):
  dsum = deg_ref[0, 0, :] + deg_ref[0, 1, :] + 1.0
  dinv = lax.rsqrt(dsum)
  tot = acc_ref[...] + g_ref[...]
  out_ref[...] = jnp.maximum(tot * dinv[:, None] + b_ref[...][None, :], 0.0)


def _finalize(acc, g, deg, b):
  deg3 = deg.reshape(NC, GRID, BLK).transpose(1, 0, 2)
  return pl.pallas_call(
      _final_body,
      grid=(GRID,),
      in_specs=[
          pl.BlockSpec((BLK, D), lambda i: (i, 0)),
          pl.BlockSpec((BLK, D), lambda i: (i, 0)),
          pl.BlockSpec((1, NC, BLK), lambda i: (i, 0, 0)),
          pl.BlockSpec((D,), lambda i: (0,)),
      ],
      out_specs=pl.BlockSpec((BLK, D), lambda i: (i, 0)),
      out_shape=jax.ShapeDtypeStruct((N, D), jnp.float32),
  )(acc, g, deg3, b)


def kernel(x, edge_index, W, b):
  E = edge_index.shape[1]
  ei = edge_index.astype(jnp.int32)

  e_deg = -(-E // (NW * 16)) * (NW * 16)
  dst_deg = jnp.concatenate(
      [ei[1], jnp.full((e_deg - E,), NPAD - 1, jnp.int32)])
  deg = _make_deg_kernel(e_deg // NW)(dst_deg)

  g = _matmul_scaled(x, W, deg)

  chunks_per_tile = -(-E // (NS * CHUNK))
  chunks_per_tile = -(-chunks_per_tile // NBUF) * NBUF
  e_pad = NS * chunks_per_tile * CHUNK
  pad = jnp.full((e_pad - E,), -1, jnp.int32)
  src = jnp.concatenate([ei[0], pad])
  dst = jnp.concatenate([ei[1], pad])
  half_id = dst // HALF
  core = jnp.arange(NC, dtype=jnp.int32)[:, None]
  own = (half_id[None, :] == core) & (dst[None, :] >= 0)
  src_m = jnp.where(own, src[None, :], -1)
  dst_m = jnp.where(own, dst[None, :] - core * HALF, -1)
  src4 = src_m.reshape(NC, NS, chunks_per_tile, CHUNK)
  dst4 = dst_m.reshape(NC, NS, chunks_per_tile, CHUNK)

  zeros = jnp.zeros((OUT_ROWS, D), jnp.float32)
  acc = _make_agg_kernel(chunks_per_tile)(g, src4, dst4, zeros)
  return _finalize(acc, g, deg, b)
